# baseline (speedup 1.0000x reference)
import jax
import jax.numpy as jnp
from jax.experimental import pallas as pl
from jax.experimental.pallas import tpu as pltpu


def _emb_kernel(x_ref, w_ref, b_ref, o_ref):
    o_ref[...] = x_ref[...] @ w_ref[...] + b_ref[...]


def _pool_kernel(h_ref, w_ref, b_ref, o_ref):
    o_ref[...] = jnp.maximum(h_ref[...] @ w_ref[...] + b_ref[...], 0.0)


def _apply_kernel(h_ref, agg_ref, w1_ref, w2_ref, b_ref, o_ref):
    t = h_ref[...] @ w1_ref[...] + agg_ref[...] @ w2_ref[...] + b_ref[...]
    nrm = jnp.sqrt(jnp.sum(t * t, axis=1, keepdims=True))
    t = t / jnp.maximum(nrm, 1e-12)
    o_ref[...] = h_ref[...] + jnp.maximum(t, 0.0)


def _readout_kernel(n_total, h_ref, w0_ref, b0_ref, w1_ref, b1_ref,
                    w2_ref, b2_ref, o_ref, acc_ref):
    i = pl.program_id(0)

    @pl.when(i == 0)
    def _init():
        acc_ref[...] = jnp.zeros_like(acc_ref)

    acc_ref[...] += jnp.sum(h_ref[...], axis=0, keepdims=True)

    @pl.when(i == pl.num_programs(0) - 1)
    def _fin():
        hg = acc_ref[...] * (1.0 / n_total)
        x = jnp.maximum(hg @ w0_ref[...] + b0_ref[...], 0.0)
        x = jnp.maximum(x @ w1_ref[...] + b1_ref[...], 0.0)
        o_ref[...] = x @ w2_ref[...] + b2_ref[...]


def kernel(graphs, nodes_feat, edges_feat, nodes_num_norm_sqrt,
           edges_num_norm_sqrt, W_emb, b_emb, W_pool, b_pool, W_node,
           b_node, W0, b0, W1, b1, W2, b2):
    src = graphs[0]
    dst = graphs[1]
    n, in_dim = nodes_feat.shape
    h_dim = W_emb.shape[1]
    n_layers = W_pool.shape[0]
    e = src.shape[0]

    bn = 2000
    nb = n // bn

    row_spec = pl.BlockSpec((bn, h_dim), lambda i: (i, 0))

    def full(a):
        return pl.BlockSpec(a.shape, lambda i: (0,) * a.ndim)

    b_emb2 = b_emb[None, :]
    h = pl.pallas_call(
        _emb_kernel,
        grid=(nb,),
        in_specs=[pl.BlockSpec((bn, in_dim), lambda i: (i, 0)),
                  full(W_emb), full(b_emb2)],
        out_specs=row_spec,
        out_shape=jax.ShapeDtypeStruct((n, h_dim), jnp.float32),
    )(nodes_feat, W_emb, b_emb2)

    deg = jax.ops.segment_sum(jnp.ones((e,), jnp.float32), dst,
                              num_segments=n)
    deg = jnp.maximum(deg, 1.0)[:, None]

    for l in range(n_layers):
        wp = W_pool[l]
        bp = b_pool[l][None, :]
        q = pl.pallas_call(
            _pool_kernel,
            grid=(nb,),
            in_specs=[row_spec, full(wp), full(bp)],
            out_specs=row_spec,
            out_shape=jax.ShapeDtypeStruct((n, h_dim), jnp.float32),
        )(h, wp, bp)
        agg = jax.ops.segment_sum(q[src], dst, num_segments=n) / deg
        w1 = W_node[l][:h_dim]
        w2 = W_node[l][h_dim:]
        bnd = b_node[l][None, :]
        h = pl.pallas_call(
            _apply_kernel,
            grid=(nb,),
            in_specs=[row_spec, row_spec, full(w1), full(w2), full(bnd)],
            out_specs=row_spec,
            out_shape=jax.ShapeDtypeStruct((n, h_dim), jnp.float32),
        )(h, agg, w1, w2, bnd)

    b0r = b0[None, :]
    b1r = b1[None, :]
    b2r = b2[None, :]
    nc = W2.shape[1]
    logits = pl.pallas_call(
        lambda *a: _readout_kernel(float(n), *a),
        grid=(nb,),
        in_specs=[row_spec, full(W0), full(b0r), full(W1), full(b1r),
                  full(W2), full(b2r)],
        out_specs=pl.BlockSpec((1, nc), lambda i: (0, 0)),
        out_shape=jax.ShapeDtypeStruct((1, nc), jnp.float32),
        scratch_shapes=[pltpu.VMEM((1, h_dim), jnp.float32)],
    )(h, W0, b0r, W1, b1r, W2, b2r)
    return logits
